# baseline (speedup 1.0000x reference)
import jax
import jax.numpy as jnp
from jax.experimental import pallas as pl
from jax.experimental.pallas import tpu as pltpu


def _make_body(n_cells):
    def body(bidx_ref, fm_ref, mask_ref, out_ref):
        neg = jnp.finfo(jnp.float32).min

        def cell(i, carry):
            b = bidx_ref[i]
            f = fm_ref[b, 0]
            m = mask_ref[i]
            masked = jnp.where((m != 0)[None, :, :], f, neg)
            out_ref[0, i, :] = jnp.max(masked, axis=(1, 2))
            return carry

        jax.lax.fori_loop(0, n_cells, cell, 0)

    return body


def kernel(feature_maps, cell_masks, cell_counts):
    B, C, H, W = feature_maps.shape
    n_cells = cell_masks.shape[0]
    HW = H * W
    C_SPLIT = 4
    C_BLK = C // C_SPLIT

    fm = feature_maps.reshape(B, C_SPLIT, C_BLK, HW // 128, 128)
    masks = cell_masks.reshape(n_cells, HW // 128, 128).astype(jnp.int8)

    ends = jnp.cumsum(cell_counts)
    batch_idx = jnp.searchsorted(ends, jnp.arange(n_cells), side="right")
    batch_idx = batch_idx.astype(jnp.int32)

    grid_spec = pltpu.PrefetchScalarGridSpec(
        num_scalar_prefetch=1,
        grid=(C_SPLIT,),
        in_specs=[
            pl.BlockSpec(
                (B, 1, C_BLK, HW // 128, 128),
                lambda c, bidx: (0, c, 0, 0, 0),
            ),
            pl.BlockSpec(
                (n_cells, HW // 128, 128),
                lambda c, bidx: (0, 0, 0),
            ),
        ],
        out_specs=pl.BlockSpec(
            (1, n_cells, C_BLK), lambda c, bidx: (c, 0, 0)
        ),
    )

    out = pl.pallas_call(
        _make_body(n_cells),
        grid_spec=grid_spec,
        out_shape=jax.ShapeDtypeStruct((C_SPLIT, n_cells, C_BLK), feature_maps.dtype),
    )(batch_idx, fm, masks)
    return out.transpose(1, 0, 2).reshape(n_cells, C)


# speedup vs baseline: 1.1092x; 1.1092x over previous
import jax
import jax.numpy as jnp
from jax.experimental import pallas as pl
from jax.experimental.pallas import tpu as pltpu

_G = 8


def _make_body(c_blk):
    def body(bidx_ref, fm_ref, mask_ref, out_ref):
        g = pl.program_id(1)
        neg = jnp.finfo(jnp.float32).min
        results = []
        for j in range(_G):
            b = bidx_ref[g * _G + j]
            f = fm_ref[b, 0]
            m = mask_ref[0, j]
            masked = jnp.where((m != 0)[None, :, :], f, neg)
            results.append(jnp.max(masked, axis=(1, 2)))
        out_ref[0, 0] = jnp.stack(results)

    return body


def kernel(feature_maps, cell_masks, cell_counts):
    B, C, H, W = feature_maps.shape
    n_cells = cell_masks.shape[0]
    HW = H * W
    C_SPLIT = 4
    C_BLK = C // C_SPLIT
    n_groups = n_cells // _G

    fm = feature_maps.reshape(B, C_SPLIT, C_BLK, HW // 128, 128)
    masks = cell_masks.reshape(n_groups, _G, HW // 128, 128).astype(jnp.int8)

    ends = jnp.cumsum(cell_counts)
    batch_idx = jnp.searchsorted(ends, jnp.arange(n_cells), side="right")
    batch_idx = jnp.minimum(batch_idx, B - 1).astype(jnp.int32)

    grid_spec = pltpu.PrefetchScalarGridSpec(
        num_scalar_prefetch=1,
        grid=(C_SPLIT, n_groups),
        in_specs=[
            pl.BlockSpec(
                (B, 1, C_BLK, HW // 128, 128),
                lambda c, g, bidx: (0, c, 0, 0, 0),
            ),
            pl.BlockSpec(
                (1, _G, HW // 128, 128),
                lambda c, g, bidx: (g, 0, 0, 0),
            ),
        ],
        out_specs=pl.BlockSpec(
            (1, 1, _G, C_BLK), lambda c, g, bidx: (c, g, 0, 0)
        ),
    )

    out = pl.pallas_call(
        _make_body(C_BLK),
        grid_spec=grid_spec,
        out_shape=jax.ShapeDtypeStruct(
            (C_SPLIT, n_groups, _G, C_BLK), feature_maps.dtype
        ),
    )(batch_idx, fm, masks)
    return out.transpose(1, 2, 0, 3).reshape(n_cells, C)
